# Initial kernel scaffold; baseline (speedup 1.0000x reference)
import jax
import jax.numpy as jnp
from jax.experimental import pallas as pl


def kernel(edge_index, edge_type, batch_vec, batch_size, max_seq_len, graph_node_offsets, edge_embedding):
    raise NotImplementedError("write your pallas kernel here")


# speedup vs baseline: 66.3898x; 66.3898x over previous
import jax
import jax.numpy as jnp
from jax.experimental import pallas as pl
from jax.experimental.pallas import tpu as pltpu


def _expand_kernel(tbl_ref, emb_ref, bv_ref, out_ref, v_ref):
    b = pl.program_id(0)

    @pl.when(b == 0)
    def _():
        p = jnp.max(tbl_ref[...], axis=0)
        t = jnp.where(p >= 0, jnp.bitwise_and(p, 15), 16)
        for k in range(16):
            m = t == k
            for h in range(16):
                if k == 0:
                    v_ref[h] = jnp.where(m, emb_ref[k, h], 0.0)
                else:
                    v_ref[h] = jnp.where(m, emb_ref[k, h], v_ref[h])

    bm = bv_ref[...] == b
    mask = jnp.broadcast_to(bm, (256, 256))
    for h in range(16):
        out_ref[0, h] = jnp.where(mask, v_ref[h], 0.0)


def _expand(tables, emb, batch_vec, B, S, H, interpret=False):
    ntbl = tables.shape[0]
    return pl.pallas_call(
        _expand_kernel,
        grid=(B,),
        in_specs=[
            pl.BlockSpec((ntbl, S, S), lambda b: (0, 0, 0)),
            pl.BlockSpec(memory_space=pltpu.SMEM),
            pl.BlockSpec((S, 1), lambda b: (0, 0)),
        ],
        out_specs=pl.BlockSpec((1, H, S, S), lambda b: (b, 0, 0, 0)),
        out_shape=jax.ShapeDtypeStruct((B, H, S, S), jnp.float32),
        scratch_shapes=[pltpu.VMEM((H, S, S), jnp.float32)],
        interpret=interpret,
    )(tables, emb, batch_vec)


def kernel(edge_index, edge_type, batch_vec, batch_size, max_seq_len,
           graph_node_offsets, edge_embedding):
    E = edge_type.shape[0]
    S = batch_vec.shape[0]
    B = graph_node_offsets.shape[0]
    H = edge_embedding.shape[1]

    src = edge_index[0]
    dst = edge_index[1]
    flat = src * S + dst
    eid = jax.lax.iota(jnp.int32, E)
    packed = jnp.bitwise_or(eid << 4, edge_type)

    tbl = jnp.full((S * S,), -1, jnp.int32).at[flat].max(packed)
    tables = tbl.reshape(1, S, S)

    bv = batch_vec.reshape(S, 1)
    return _expand(tables, edge_embedding, bv, B, S, H)


# speedup vs baseline: 322.3838x; 4.8559x over previous
import functools

import jax
import jax.numpy as jnp
from jax import lax
from jax.experimental import pallas as pl
from jax.experimental.pallas import tpu as pltpu
from jax.experimental.pallas import tpu_sc as plsc

_NC = 2
_NS = 16
_NW = _NC * _NS
_L = 16


def _sc_scatter_body(flat_hbm, packed_hbm, out_hbm, flat_v, packed_v, tbl_v,
                     *, epw, slots):
    wid = lax.axis_index("s") * _NC + lax.axis_index("c")
    base = wid * epw
    pltpu.sync_copy(flat_hbm.at[pl.ds(base, epw)], flat_v)
    pltpu.sync_copy(packed_hbm.at[pl.ds(base, epw)], packed_v)

    neg1 = jnp.full((_L,), -1, jnp.int32)

    def init_body(i, _):
        for u in range(8):
            tbl_v[pl.ds((i * 8 + u) * _L, _L)] = neg1
        return 0

    lax.fori_loop(0, slots // (_L * 8), init_body, 0)

    def edge_body(i, _):
        for u in range(4):
            sl = pl.ds((i * 4 + u) * _L, _L)
            fl = flat_v[sl]
            pk = packed_v[sl]
            _, keep = plsc.scan_count(fl)
            plsc.store_scatter(tbl_v, [fl], pk, mask=keep)
        return 0

    lax.fori_loop(0, epw // (_L * 4), edge_body, 0)

    pltpu.sync_copy(tbl_v, out_hbm.at[wid])


def _sc_scatter(flat, packed, S):
    E = flat.shape[0]
    epw = E // _NW
    slots = S * S
    mesh = plsc.VectorSubcoreMesh(core_axis_name="c", subcore_axis_name="s")
    body = functools.partial(_sc_scatter_body, epw=epw, slots=slots)
    return pl.kernel(
        body,
        out_type=jax.ShapeDtypeStruct((_NW, slots), jnp.int32),
        mesh=mesh,
        scratch_types=[
            pltpu.VMEM((epw,), jnp.int32),
            pltpu.VMEM((epw,), jnp.int32),
            pltpu.VMEM((slots,), jnp.int32),
        ],
        compiler_params=pltpu.CompilerParams(needs_layout_passes=False),
    )(flat, packed)


def _expand_kernel(tbl_ref, emb_ref, bv_ref, out_ref, v_ref):
    b = pl.program_id(0)

    @pl.when(b == 0)
    def _():
        p = jnp.max(tbl_ref[...], axis=0)
        t = jnp.where(p >= 0, jnp.bitwise_and(p, 15), 16)
        for k in range(16):
            m = t == k
            for h in range(16):
                if k == 0:
                    v_ref[h] = jnp.where(m, emb_ref[k, h], 0.0)
                else:
                    v_ref[h] = jnp.where(m, emb_ref[k, h], v_ref[h])

    bm = bv_ref[...] == b
    mask = jnp.broadcast_to(bm, out_ref.shape[2:])
    for h in range(16):
        out_ref[0, h] = jnp.where(mask, v_ref[h], 0.0)


def _expand(tables, emb, batch_vec, B, S, H):
    ntbl = tables.shape[0]
    return pl.pallas_call(
        _expand_kernel,
        grid=(B,),
        in_specs=[
            pl.BlockSpec((ntbl, S, S), lambda b: (0, 0, 0)),
            pl.BlockSpec(memory_space=pltpu.SMEM),
            pl.BlockSpec((S, 1), lambda b: (0, 0)),
        ],
        out_specs=pl.BlockSpec((1, H, S, S), lambda b: (b, 0, 0, 0)),
        out_shape=jax.ShapeDtypeStruct((B, H, S, S), jnp.float32),
        scratch_shapes=[pltpu.VMEM((H, S, S), jnp.float32)],
    )(tables, emb, batch_vec)


def kernel(edge_index, edge_type, batch_vec, batch_size, max_seq_len,
           graph_node_offsets, edge_embedding):
    E = edge_type.shape[0]
    S = batch_vec.shape[0]
    B = graph_node_offsets.shape[0]
    H = edge_embedding.shape[1]

    src = edge_index[0]
    dst = edge_index[1]
    flat = src * S + dst
    eid = lax.iota(jnp.int32, E)
    packed = jnp.bitwise_or(eid << 4, edge_type)

    tables = _sc_scatter(flat, packed, S).reshape(_NW, S, S)

    bv = batch_vec.reshape(S, 1)
    return _expand(tables, edge_embedding, bv, B, S, H)
